# baseline (speedup 1.0000x reference)
import functools

import jax
import jax.numpy as jnp
from jax.experimental import pallas as pl
from jax.experimental.pallas import tpu as pltpu
from jax.experimental.pallas import tpu_sc as plsc

_N = 10000
_E = 320000
_D = 128
_H = 16
_C = 2

_NC = 2
_NS = 16
_NW = _NC * _NS
_L = 16

_ET = _E // _NW
_K = 80
_NCHUNK = _ET // _K
_NPAD = 10240
_RPT = _NPAD // _NS


def _sc_mesh():
    return plsc.VectorSubcoreMesh(core_axis_name="c", subcore_axis_name="s")


_SC_PARAMS = pltpu.CompilerParams(use_tc_tiling_on_sc=False)


_NBUF = 4


def _sc_degree(dst2d):
    @functools.partial(
        pl.kernel,
        out_type=jax.ShapeDtypeStruct((_NC, _NPAD, _L), jnp.float32),
        mesh=_sc_mesh(),
        compiler_params=_SC_PARAMS,
        scratch_types=[
            pltpu.VMEM((_NCHUNK, _K), jnp.int32),
            pltpu.VMEM((_K, _L), jnp.float32),
            pltpu.VMEM((_RPT, _L), jnp.float32),
            pltpu.VMEM_SHARED((_NPAD, _L), jnp.float32),
            pltpu.SemaphoreType.DMA,
        ],
    )
    def deg_kernel(dst_hbm, out_hbm, didx, ones, stage, acc, dsem):
        c = jax.lax.axis_index("c")
        s = jax.lax.axis_index("s")
        wid = s * _NC + c

        @pl.loop(0, _RPT)
        def _(i):
            stage[i] = jnp.zeros((_L,), jnp.float32)

        @pl.loop(0, _K)
        def _(i):
            ones[i] = jnp.ones((_L,), jnp.float32)

        pltpu.sync_copy(stage, acc.at[pl.ds(s * _RPT, _RPT)])
        pltpu.sync_copy(dst_hbm.at[pl.ds(wid * _NCHUNK, _NCHUNK)], didx)
        plsc.subcore_barrier()

        @pl.loop(0, _NCHUNK - (_NCHUNK % _NBUF), step=_NBUF)
        def _(g):
            for b in range(_NBUF):
                pltpu.async_copy(ones, acc.at[didx.at[g + b]], dsem, add=True)
            for b in range(_NBUF):
                pltpu.make_async_copy(ones, acc.at[didx.at[g + b]], dsem).wait()

        for b in range(_NCHUNK % _NBUF):
            jj = _NCHUNK - (_NCHUNK % _NBUF) + b
            pltpu.sync_copy(ones, acc.at[didx.at[jj]], add=True)

        plsc.subcore_barrier()
        pltpu.sync_copy(acc.at[pl.ds(s * _RPT, _RPT)], stage)
        pltpu.sync_copy(stage, out_hbm.at[c].at[pl.ds(s * _RPT, _RPT)])

    return deg_kernel(dst2d)


def _sc_aggregate(table, src2d, dst2d):
    @functools.partial(
        pl.kernel,
        out_type=jax.ShapeDtypeStruct((_NC, _NPAD, _L), jnp.float32),
        mesh=_sc_mesh(),
        compiler_params=_SC_PARAMS,
        scratch_types=[
            pltpu.VMEM((_NCHUNK, _K), jnp.int32),
            pltpu.VMEM((_NCHUNK, _K), jnp.int32),
            pltpu.VMEM((_NBUF, _K, _L), jnp.float32),
            pltpu.VMEM((_RPT, _L), jnp.float32),
            pltpu.VMEM_SHARED((_NPAD, _L), jnp.float32),
        ] + [pltpu.SemaphoreType.DMA] * (_NBUF + 1),
    )
    def agg_kernel(table_hbm, src_hbm, dst_hbm, out_hbm,
                   sidx, didx, rows, stage, acc, *sems):
        gsems, ssem = sems[:_NBUF], sems[_NBUF]
        c = jax.lax.axis_index("c")
        s = jax.lax.axis_index("s")
        wid = s * _NC + c

        @pl.loop(0, _RPT)
        def _(i):
            stage[i] = jnp.zeros((_L,), jnp.float32)

        pltpu.sync_copy(src_hbm.at[pl.ds(wid * _NCHUNK, _NCHUNK)], sidx)
        pltpu.sync_copy(dst_hbm.at[pl.ds(wid * _NCHUNK, _NCHUNK)], didx)
        pltpu.sync_copy(stage, acc.at[pl.ds(s * _RPT, _RPT)])
        plsc.subcore_barrier()

        for b in range(_NBUF):
            pltpu.async_copy(table_hbm.at[sidx.at[b]], rows.at[b], gsems[b])

        @pl.loop(0, _NCHUNK - (_NCHUNK % _NBUF), step=_NBUF)
        def _(g):
            for b in range(_NBUF):
                pltpu.make_async_copy(
                    table_hbm.at[sidx.at[g + b]], rows.at[b], gsems[b]).wait()
            for b in range(_NBUF):
                pltpu.async_copy(rows.at[b], acc.at[didx.at[g + b]], ssem,
                                 add=True)
            for b in range(_NBUF):
                pltpu.make_async_copy(rows.at[b], acc.at[didx.at[g + b]],
                                      ssem).wait()
            for b in range(_NBUF):
                nxt = g + b + _NBUF

                @pl.when(nxt < _NCHUNK)
                def _():
                    pltpu.async_copy(
                        table_hbm.at[sidx.at[nxt]], rows.at[b], gsems[b])

        for b in range(_NCHUNK % _NBUF):
            jj = _NCHUNK - (_NCHUNK % _NBUF) + b
            pltpu.make_async_copy(
                table_hbm.at[sidx.at[jj]], rows.at[b], gsems[b]).wait()
            pltpu.sync_copy(rows.at[b], acc.at[didx.at[jj]], add=True)

        plsc.subcore_barrier()
        pltpu.sync_copy(acc.at[pl.ds(s * _RPT, _RPT)], stage)
        pltpu.sync_copy(stage, out_hbm.at[c].at[pl.ds(s * _RPT, _RPT)])

    return agg_kernel(table, src2d, dst2d)


def _tc_scale(degp, x, W1):
    def body(degp_ref, x_ref, w_ref, dis_ref, xw_ref, y1_ref):
        xw = jnp.dot(x_ref[...], w_ref[...],
                     preferred_element_type=jnp.float32)
        deg = degp_ref[0] + degp_ref[1] + 1.0
        dis = jax.lax.rsqrt(deg)
        dis_ref[...] = dis
        xw_ref[...] = xw
        y1_ref[...] = dis * xw

    return pl.pallas_call(
        body,
        out_shape=(
            jax.ShapeDtypeStruct((_N, _H), jnp.float32),
            jax.ShapeDtypeStruct((_N, _H), jnp.float32),
            jax.ShapeDtypeStruct((_N, _H), jnp.float32),
        ),
    )(degp, x, W1)


def _tc_layer1(agg1p, dis, xw, b1):
    def body(aggp_ref, dis_ref, xw_ref, b1_ref, h_ref, y2_ref):
        dis = dis_ref[...]
        agg = aggp_ref[0] + aggp_ref[1]
        pre = dis * agg + dis * dis * xw_ref[...] + b1_ref[...]
        h = jnp.maximum(pre, 0.0)
        h_ref[...] = h
        y2_ref[...] = dis * h

    return pl.pallas_call(
        body,
        out_shape=(
            jax.ShapeDtypeStruct((_N, _H), jnp.float32),
            jax.ShapeDtypeStruct((_N, _H), jnp.float32),
        ),
    )(agg1p, dis, xw, b1.reshape(1, _H))


def _tc_out(agg2p, dis, h, W2, b2):
    def body(aggp_ref, dis_ref, h_ref, w2_ref, b2_ref, o_ref):
        dis = dis_ref[...]
        agg = aggp_ref[0] + aggp_ref[1]
        t = dis * agg + dis * dis * h_ref[...]
        o = jnp.dot(t, w2_ref[...], preferred_element_type=jnp.float32)
        o = o + b2_ref[...]
        m = jnp.max(o, axis=1, keepdims=True)
        lse = m + jnp.log(jnp.sum(jnp.exp(o - m), axis=1, keepdims=True))
        o_ref[...] = o - lse

    return pl.pallas_call(
        body,
        out_shape=jax.ShapeDtypeStruct((_N, _C), jnp.float32),
    )(agg2p, dis, h, W2, b2.reshape(1, _C))


def kernel(x, edge_index, W1, b1, W2, b2):
    src2d = edge_index[0].astype(jnp.int32).reshape(_NW * _NCHUNK, _K)
    dst2d = edge_index[1].astype(jnp.int32).reshape(_NW * _NCHUNK, _K)

    degp = _sc_degree(dst2d)[:, :_N]
    dis, xw, y1 = _tc_scale(degp, x, W1)
    agg1p = _sc_aggregate(y1, src2d, dst2d)[:, :_N]
    h, y2 = _tc_layer1(agg1p, dis, xw, b1)
    agg2p = _sc_aggregate(y2, src2d, dst2d)[:, :_N]
    return _tc_out(agg2p, dis, h, W2, b2)


# speedup vs baseline: 1.0984x; 1.0984x over previous
import functools

import jax
import jax.numpy as jnp
from jax.experimental import pallas as pl
from jax.experimental.pallas import tpu as pltpu
from jax.experimental.pallas import tpu_sc as plsc

_N = 10000
_E = 320000
_D = 128
_H = 16
_C = 2

_NC = 2
_NS = 16
_NW = _NC * _NS
_L = 16

_ET = _E // _NW
_K = 80
_NCHUNK = _ET // _K
_NPAD = 10240
_RPT = _NPAD // _NS


def _sc_mesh():
    return plsc.VectorSubcoreMesh(core_axis_name="c", subcore_axis_name="s")


_SC_PARAMS = pltpu.CompilerParams(use_tc_tiling_on_sc=False)


_NBUF = 4
_LAG = 4
_NRING = 2 * _LAG


def _sc_degree(dst2d):
    @functools.partial(
        pl.kernel,
        out_type=jax.ShapeDtypeStruct((_NC, _NPAD, _L), jnp.float32),
        mesh=_sc_mesh(),
        compiler_params=_SC_PARAMS,
        scratch_types=[
            pltpu.VMEM((_NCHUNK, _K), jnp.int32),
            pltpu.VMEM((_K, _L), jnp.float32),
            pltpu.VMEM((_RPT, _L), jnp.float32),
            pltpu.VMEM_SHARED((_NPAD, _L), jnp.float32),
            pltpu.SemaphoreType.DMA,
        ],
    )
    def deg_kernel(dst_hbm, out_hbm, didx, ones, stage, acc, dsem):
        c = jax.lax.axis_index("c")
        s = jax.lax.axis_index("s")
        wid = s * _NC + c

        @pl.loop(0, _RPT)
        def _(i):
            stage[i] = jnp.zeros((_L,), jnp.float32)

        @pl.loop(0, _K)
        def _(i):
            ones[i] = jnp.ones((_L,), jnp.float32)

        pltpu.sync_copy(stage, acc.at[pl.ds(s * _RPT, _RPT)])
        pltpu.sync_copy(dst_hbm.at[pl.ds(wid * _NCHUNK, _NCHUNK)], didx)
        plsc.subcore_barrier()

        @pl.loop(0, _NCHUNK - (_NCHUNK % _NBUF), step=_NBUF)
        def _(g):
            for b in range(_NBUF):
                pltpu.async_copy(ones, acc.at[didx.at[g + b]], dsem, add=True)
            for b in range(_NBUF):
                pltpu.make_async_copy(ones, acc.at[didx.at[g + b]], dsem).wait()

        for b in range(_NCHUNK % _NBUF):
            jj = _NCHUNK - (_NCHUNK % _NBUF) + b
            pltpu.sync_copy(ones, acc.at[didx.at[jj]], add=True)

        plsc.subcore_barrier()
        pltpu.sync_copy(acc.at[pl.ds(s * _RPT, _RPT)], stage)
        pltpu.sync_copy(stage, out_hbm.at[c].at[pl.ds(s * _RPT, _RPT)])

    return deg_kernel(dst2d)


def _sc_aggregate(table, src2d, dst2d):
    @functools.partial(
        pl.kernel,
        out_type=jax.ShapeDtypeStruct((_NC, _NPAD, _L), jnp.float32),
        mesh=_sc_mesh(),
        compiler_params=_SC_PARAMS,
        scratch_types=[
            pltpu.VMEM((_NCHUNK, _K), jnp.int32),
            pltpu.VMEM((_NCHUNK, _K), jnp.int32),
            pltpu.VMEM((_NRING, _K, _L), jnp.float32),
            pltpu.VMEM((_RPT, _L), jnp.float32),
            pltpu.VMEM_SHARED((_NPAD, _L), jnp.float32),
        ] + [pltpu.SemaphoreType.DMA] * (2 * _NRING),
    )
    def agg_kernel(table_hbm, src_hbm, dst_hbm, out_hbm,
                   sidx, didx, rows, stage, acc, *sems):
        gsems, ssems = sems[:_NRING], sems[_NRING:]
        c = jax.lax.axis_index("c")
        s = jax.lax.axis_index("s")
        wid = s * _NC + c

        @pl.loop(0, _RPT)
        def _(i):
            stage[i] = jnp.zeros((_L,), jnp.float32)

        pltpu.sync_copy(src_hbm.at[pl.ds(wid * _NCHUNK, _NCHUNK)], sidx)
        pltpu.sync_copy(dst_hbm.at[pl.ds(wid * _NCHUNK, _NCHUNK)], didx)
        pltpu.sync_copy(stage, acc.at[pl.ds(s * _RPT, _RPT)])
        plsc.subcore_barrier()

        def step(t, b):
            bn = (b + _LAG) % _NRING
            pltpu.make_async_copy(
                table_hbm.at[sidx.at[t]], rows.at[b], gsems[b]).wait()
            pltpu.async_copy(rows.at[b], acc.at[didx.at[t]], ssems[b],
                             add=True)

            @pl.when(t >= _LAG)
            def _():
                pltpu.make_async_copy(
                    rows.at[bn], acc.at[didx.at[t]], ssems[bn]).wait()

            @pl.when(t + _LAG < _NCHUNK)
            def _():
                pltpu.async_copy(
                    table_hbm.at[sidx.at[t + _LAG]], rows.at[bn], gsems[bn])

        for b in range(_LAG):
            pltpu.async_copy(table_hbm.at[sidx.at[b]], rows.at[b], gsems[b])

        nmain = _NCHUNK - (_NCHUNK % _NRING)

        @pl.loop(0, nmain, step=_NRING)
        def _(g):
            for b in range(_NRING):
                step(g + b, b)

        for t in range(nmain, _NCHUNK):
            step(t, t % _NRING)

        for t in range(_NCHUNK - _LAG, _NCHUNK):
            b = t % _NRING
            pltpu.make_async_copy(rows.at[b], acc.at[didx.at[0]],
                                  ssems[b]).wait()

        plsc.subcore_barrier()
        pltpu.sync_copy(acc.at[pl.ds(s * _RPT, _RPT)], stage)
        pltpu.sync_copy(stage, out_hbm.at[c].at[pl.ds(s * _RPT, _RPT)])

    return agg_kernel(table, src2d, dst2d)


def _tc_scale(degp, x, W1):
    def body(degp_ref, x_ref, w_ref, dis_ref, xw_ref, y1_ref):
        xw = jnp.dot(x_ref[...], w_ref[...],
                     preferred_element_type=jnp.float32)
        deg = degp_ref[0] + degp_ref[1] + 1.0
        dis = jax.lax.rsqrt(deg)
        dis_ref[...] = dis
        xw_ref[...] = xw
        y1_ref[...] = dis * xw

    return pl.pallas_call(
        body,
        out_shape=(
            jax.ShapeDtypeStruct((_N, _H), jnp.float32),
            jax.ShapeDtypeStruct((_N, _H), jnp.float32),
            jax.ShapeDtypeStruct((_N, _H), jnp.float32),
        ),
    )(degp, x, W1)


def _tc_layer1(agg1p, dis, xw, b1):
    def body(aggp_ref, dis_ref, xw_ref, b1_ref, h_ref, y2_ref):
        dis = dis_ref[...]
        agg = aggp_ref[0] + aggp_ref[1]
        pre = dis * agg + dis * dis * xw_ref[...] + b1_ref[...]
        h = jnp.maximum(pre, 0.0)
        h_ref[...] = h
        y2_ref[...] = dis * h

    return pl.pallas_call(
        body,
        out_shape=(
            jax.ShapeDtypeStruct((_N, _H), jnp.float32),
            jax.ShapeDtypeStruct((_N, _H), jnp.float32),
        ),
    )(agg1p, dis, xw, b1.reshape(1, _H))


def _tc_out(agg2p, dis, h, W2, b2):
    def body(aggp_ref, dis_ref, h_ref, w2_ref, b2_ref, o_ref):
        dis = dis_ref[...]
        agg = aggp_ref[0] + aggp_ref[1]
        t = dis * agg + dis * dis * h_ref[...]
        o = jnp.dot(t, w2_ref[...], preferred_element_type=jnp.float32)
        o = o + b2_ref[...]
        m = jnp.max(o, axis=1, keepdims=True)
        lse = m + jnp.log(jnp.sum(jnp.exp(o - m), axis=1, keepdims=True))
        o_ref[...] = o - lse

    return pl.pallas_call(
        body,
        out_shape=jax.ShapeDtypeStruct((_N, _C), jnp.float32),
    )(agg2p, dis, h, W2, b2.reshape(1, _C))


def kernel(x, edge_index, W1, b1, W2, b2):
    src2d = edge_index[0].astype(jnp.int32).reshape(_NW * _NCHUNK, _K)
    dst2d = edge_index[1].astype(jnp.int32).reshape(_NW * _NCHUNK, _K)

    degp = _sc_degree(dst2d)[:, :_N]
    dis, xw, y1 = _tc_scale(degp, x, W1)
    agg1p = _sc_aggregate(y1, src2d, dst2d)[:, :_N]
    h, y2 = _tc_layer1(agg1p, dis, xw, b1)
    agg2p = _sc_aggregate(y2, src2d, dst2d)[:, :_N]
    return _tc_out(agg2p, dis, h, W2, b2)


# speedup vs baseline: 1.2140x; 1.1053x over previous
import functools

import jax
import jax.numpy as jnp
from jax.experimental import pallas as pl
from jax.experimental.pallas import tpu as pltpu
from jax.experimental.pallas import tpu_sc as plsc

_N = 10000
_E = 320000
_D = 128
_H = 16
_C = 2

_NC = 2
_NS = 16
_NW = _NC * _NS
_L = 16

_ET = _E // _NW
_K = 80
_NCHUNK = _ET // _K
_NPAD = 10240
_RPT = _NPAD // _NS


def _sc_mesh():
    return plsc.VectorSubcoreMesh(core_axis_name="c", subcore_axis_name="s")


_SC_PARAMS = pltpu.CompilerParams(use_tc_tiling_on_sc=False)


_NBUF = 4
_LAG = 4
_NRING = 2 * _LAG


def _sc_degree(dst2d):
    @functools.partial(
        pl.kernel,
        out_type=jax.ShapeDtypeStruct((_NC, _NPAD, _L), jnp.float32),
        mesh=_sc_mesh(),
        compiler_params=_SC_PARAMS,
        scratch_types=[
            pltpu.VMEM((_NCHUNK, _K), jnp.int32),
            pltpu.VMEM((_K, _L), jnp.float32),
            pltpu.VMEM((_RPT, _L), jnp.float32),
            pltpu.VMEM_SHARED((_NPAD, _L), jnp.float32),
            pltpu.SemaphoreType.DMA,
        ],
    )
    def deg_kernel(dst_hbm, out_hbm, didx, ones, stage, acc, dsem):
        c = jax.lax.axis_index("c")
        s = jax.lax.axis_index("s")
        wid = s * _NC + c

        @pl.loop(0, _RPT)
        def _(i):
            stage[i] = jnp.zeros((_L,), jnp.float32)

        @pl.loop(0, _K)
        def _(i):
            ones[i] = jnp.ones((_L,), jnp.float32)

        pltpu.sync_copy(stage, acc.at[pl.ds(s * _RPT, _RPT)])
        pltpu.sync_copy(dst_hbm.at[pl.ds(wid * _NCHUNK, _NCHUNK)], didx)
        plsc.subcore_barrier()

        @pl.loop(0, _NCHUNK - (_NCHUNK % _NBUF), step=_NBUF)
        def _(g):
            for b in range(_NBUF):
                pltpu.async_copy(ones, acc.at[didx.at[g + b]], dsem, add=True)
            for b in range(_NBUF):
                pltpu.make_async_copy(ones, acc.at[didx.at[g + b]], dsem).wait()

        for b in range(_NCHUNK % _NBUF):
            jj = _NCHUNK - (_NCHUNK % _NBUF) + b
            pltpu.sync_copy(ones, acc.at[didx.at[jj]], add=True)

        plsc.subcore_barrier()
        pltpu.sync_copy(acc.at[pl.ds(s * _RPT, _RPT)], stage)
        pltpu.sync_copy(stage, out_hbm.at[c].at[pl.ds(s * _RPT, _RPT)])

    return deg_kernel(dst2d)


def _sc_aggregate(table, src2d, dst2d):
    @functools.partial(
        pl.kernel,
        out_type=jax.ShapeDtypeStruct((_NC, _NPAD, _L), jnp.float32),
        mesh=_sc_mesh(),
        compiler_params=_SC_PARAMS,
        scratch_types=[
            pltpu.VMEM((_NCHUNK, _K), jnp.int32),
            pltpu.VMEM((_NCHUNK, _K), jnp.int32),
            pltpu.VMEM((_NRING, _K, _L), jnp.float32),
            pltpu.VMEM((_RPT, _L), jnp.float32),
            pltpu.VMEM_SHARED((_NPAD, _L), jnp.float32),
        ] + [pltpu.SemaphoreType.DMA] * (2 * _NRING),
    )
    def agg_kernel(table_hbm, src_hbm, dst_hbm, out_hbm,
                   sidx, didx, rows, stage, acc, *sems):
        gsems, ssems = sems[:_NRING], sems[_NRING:]
        c = jax.lax.axis_index("c")
        s = jax.lax.axis_index("s")
        wid = s * _NC + c

        @pl.loop(0, _RPT)
        def _(i):
            stage[i] = jnp.zeros((_L,), jnp.float32)

        pltpu.sync_copy(src_hbm.at[pl.ds(wid * _NCHUNK, _NCHUNK)], sidx)
        pltpu.sync_copy(dst_hbm.at[pl.ds(wid * _NCHUNK, _NCHUNK)], didx)
        pltpu.sync_copy(stage, acc.at[pl.ds(s * _RPT, _RPT)])
        plsc.subcore_barrier()

        def step(t, b):
            bn = (b + _LAG) % _NRING
            pltpu.make_async_copy(
                table_hbm.at[sidx.at[t]], rows.at[b], gsems[b]).wait()
            pltpu.async_copy(rows.at[b], acc.at[didx.at[t]], ssems[b],
                             add=True)

            @pl.when(t >= _LAG)
            def _():
                pltpu.make_async_copy(
                    rows.at[bn], acc.at[didx.at[t]], ssems[bn]).wait()

            @pl.when(t + _LAG < _NCHUNK)
            def _():
                pltpu.async_copy(
                    table_hbm.at[sidx.at[t + _LAG]], rows.at[bn], gsems[bn])

        for b in range(_LAG):
            pltpu.async_copy(table_hbm.at[sidx.at[b]], rows.at[b], gsems[b])

        nmain = _NCHUNK - (_NCHUNK % _NRING)

        @pl.loop(0, nmain, step=_NRING)
        def _(g):
            for b in range(_NRING):
                step(g + b, b)

        for t in range(nmain, _NCHUNK):
            step(t, t % _NRING)

        for t in range(_NCHUNK - _LAG, _NCHUNK):
            b = t % _NRING
            pltpu.make_async_copy(rows.at[b], acc.at[didx.at[0]],
                                  ssems[b]).wait()

        plsc.subcore_barrier()
        pltpu.sync_copy(acc.at[pl.ds(s * _RPT, _RPT)], stage)
        pltpu.sync_copy(stage, out_hbm.at[c].at[pl.ds(s * _RPT, _RPT)])

    return agg_kernel(table, src2d, dst2d)


def _tc_xw(x, W1):
    def body(x_ref, w_ref, o_ref):
        o_ref[...] = jnp.dot(x_ref[...], w_ref[...],
                             preferred_element_type=jnp.float32)

    return pl.pallas_call(
        body,
        out_shape=jax.ShapeDtypeStruct((_N, _H), jnp.float32),
    )(x, W1)


def _tc_scale(degp, xw):
    def body(degp_ref, xw_ref, dis_ref, y1_ref):
        deg = degp_ref[0, :_N] + degp_ref[1, :_N] + 1.0
        dis = jax.lax.rsqrt(deg)
        dis_ref[...] = dis
        y1_ref[...] = dis * xw_ref[...]

    return pl.pallas_call(
        body,
        out_shape=(
            jax.ShapeDtypeStruct((_N, _H), jnp.float32),
            jax.ShapeDtypeStruct((_N, _H), jnp.float32),
        ),
    )(degp, xw)


def _tc_layer1(agg1p, dis, xw, b1):
    def body(aggp_ref, dis_ref, xw_ref, b1_ref, h_ref, y2_ref):
        dis = dis_ref[...]
        agg = aggp_ref[0, :_N] + aggp_ref[1, :_N]
        pre = dis * agg + dis * dis * xw_ref[...] + b1_ref[...]
        h = jnp.maximum(pre, 0.0)
        h_ref[...] = h
        y2_ref[...] = dis * h

    return pl.pallas_call(
        body,
        out_shape=(
            jax.ShapeDtypeStruct((_N, _H), jnp.float32),
            jax.ShapeDtypeStruct((_N, _H), jnp.float32),
        ),
    )(agg1p, dis, xw, b1.reshape(1, _H))


def _tc_out(agg2p, dis, h, W2, b2):
    def body(aggp_ref, dis_ref, h_ref, w2_ref, b2_ref, o_ref):
        dis = dis_ref[...]
        agg = aggp_ref[0, :_N] + aggp_ref[1, :_N]
        t = dis * agg + dis * dis * h_ref[...]
        o = jnp.dot(t, w2_ref[...], preferred_element_type=jnp.float32)
        o = o + b2_ref[...]
        m = jnp.max(o, axis=1, keepdims=True)
        lse = m + jnp.log(jnp.sum(jnp.exp(o - m), axis=1, keepdims=True))
        o_ref[...] = o - lse

    return pl.pallas_call(
        body,
        out_shape=jax.ShapeDtypeStruct((_N, _C), jnp.float32),
    )(agg2p, dis, h, W2, b2.reshape(1, _C))


def kernel(x, edge_index, W1, b1, W2, b2):
    src2d = edge_index[0].astype(jnp.int32).reshape(_NW * _NCHUNK, _K)
    dst2d = edge_index[1].astype(jnp.int32).reshape(_NW * _NCHUNK, _K)
    src2d, dst2d = jax.lax.optimization_barrier((src2d, dst2d))

    degp = _sc_degree(dst2d)
    xw = _tc_xw(x, W1)
    dis, y1 = _tc_scale(degp, xw)
    agg1p = _sc_aggregate(y1, src2d, dst2d)
    h, y2 = _tc_layer1(agg1p, dis, xw, b1)
    agg2p = _sc_aggregate(y2, src2d, dst2d)
    return _tc_out(agg2p, dis, h, W2, b2)


# speedup vs baseline: 1.7142x; 1.4120x over previous
import functools

import jax
import jax.numpy as jnp
from jax.experimental import pallas as pl
from jax.experimental.pallas import tpu as pltpu
from jax.experimental.pallas import tpu_sc as plsc

_N = 10000
_E = 320000
_D = 128
_H = 16
_C = 2

_NC = 2
_NS = 16
_NW = _NC * _NS
_L = 16

_ET = _E // _NW
_K = 80
_NCHUNK = _ET // _K
_NPAD = 10240
_RPT = _NPAD // _NS


def _sc_mesh():
    return plsc.VectorSubcoreMesh(core_axis_name="c", subcore_axis_name="s")


_SC_PARAMS = pltpu.CompilerParams(use_tc_tiling_on_sc=False)


_NBUF = 4
_LAG = 4
_NRING = 2 * _LAG


def _sc_degree(dst2d):
    @functools.partial(
        pl.kernel,
        out_type=jax.ShapeDtypeStruct((_NC, _NPAD, _L), jnp.float32),
        mesh=_sc_mesh(),
        compiler_params=_SC_PARAMS,
        scratch_types=[
            pltpu.VMEM((_NCHUNK, _K), jnp.int32),
            pltpu.VMEM((_K, _L), jnp.float32),
            pltpu.VMEM((_RPT, _L), jnp.float32),
            pltpu.VMEM_SHARED((_NPAD, _L), jnp.float32),
            pltpu.SemaphoreType.DMA,
        ],
    )
    def deg_kernel(dst_hbm, out_hbm, didx, ones, stage, acc, dsem):
        c = jax.lax.axis_index("c")
        s = jax.lax.axis_index("s")
        wid = s * _NC + c

        @pl.loop(0, _RPT)
        def _(i):
            stage[i] = jnp.zeros((_L,), jnp.float32)

        @pl.loop(0, _K)
        def _(i):
            ones[i] = jnp.ones((_L,), jnp.float32)

        pltpu.sync_copy(stage, acc.at[pl.ds(s * _RPT, _RPT)])
        pltpu.sync_copy(dst_hbm.at[pl.ds(wid * _NCHUNK, _NCHUNK)], didx)
        plsc.subcore_barrier()

        @pl.loop(0, _NCHUNK - (_NCHUNK % _NBUF), step=_NBUF)
        def _(g):
            for b in range(_NBUF):
                pltpu.async_copy(ones, acc.at[didx.at[g + b]], dsem, add=True)
            for b in range(_NBUF):
                pltpu.make_async_copy(ones, acc.at[didx.at[g + b]], dsem).wait()

        for b in range(_NCHUNK % _NBUF):
            jj = _NCHUNK - (_NCHUNK % _NBUF) + b
            pltpu.sync_copy(ones, acc.at[didx.at[jj]], add=True)

        plsc.subcore_barrier()
        pltpu.sync_copy(acc.at[pl.ds(s * _RPT, _RPT)], stage)
        pltpu.sync_copy(stage, out_hbm.at[c].at[pl.ds(s * _RPT, _RPT)])

    return deg_kernel(dst2d)


def _sc_aggregate(table, src2d, dst2d):
    @functools.partial(
        pl.kernel,
        out_type=jax.ShapeDtypeStruct((_NC, _NPAD, _L), jnp.float32),
        mesh=_sc_mesh(),
        compiler_params=_SC_PARAMS,
        scratch_types=[
            pltpu.VMEM((_NCHUNK, _K), jnp.int32),
            pltpu.VMEM((_NCHUNK, _K), jnp.int32),
            pltpu.VMEM((_NRING, _K, _L), jnp.float32),
            pltpu.VMEM((_RPT, _L), jnp.float32),
            pltpu.VMEM_SHARED((_NPAD, _L), jnp.float32),
        ] + [pltpu.SemaphoreType.DMA] * (2 * _NRING),
    )
    def agg_kernel(table_hbm, src_hbm, dst_hbm, out_hbm,
                   sidx, didx, rows, stage, acc, *sems):
        gsems, ssems = sems[:_NRING], sems[_NRING:]
        c = jax.lax.axis_index("c")
        s = jax.lax.axis_index("s")
        wid = s * _NC + c

        @pl.loop(0, _RPT)
        def _(i):
            stage[i] = jnp.zeros((_L,), jnp.float32)

        pltpu.sync_copy(src_hbm.at[pl.ds(wid * _NCHUNK, _NCHUNK)], sidx)
        pltpu.sync_copy(dst_hbm.at[pl.ds(wid * _NCHUNK, _NCHUNK)], didx)
        pltpu.sync_copy(stage, acc.at[pl.ds(s * _RPT, _RPT)])
        plsc.subcore_barrier()

        def step(t, b):
            bn = (b + _LAG) % _NRING
            pltpu.make_async_copy(
                table_hbm.at[sidx.at[t]], rows.at[b], gsems[b]).wait()
            pltpu.async_copy(rows.at[b], acc.at[didx.at[t]], ssems[b],
                             add=True)

            @pl.when(t >= _LAG)
            def _():
                pltpu.make_async_copy(
                    rows.at[bn], acc.at[didx.at[t]], ssems[bn]).wait()

            @pl.when(t + _LAG < _NCHUNK)
            def _():
                pltpu.async_copy(
                    table_hbm.at[sidx.at[t + _LAG]], rows.at[bn], gsems[bn])

        for b in range(_LAG):
            pltpu.async_copy(table_hbm.at[sidx.at[b]], rows.at[b], gsems[b])

        nmain = _NCHUNK - (_NCHUNK % _NRING)

        @pl.loop(0, nmain, step=_NRING)
        def _(g):
            for b in range(_NRING):
                step(g + b, b)

        for t in range(nmain, _NCHUNK):
            step(t, t % _NRING)

        for t in range(_NCHUNK - _LAG, _NCHUNK):
            b = t % _NRING
            pltpu.make_async_copy(rows.at[b], acc.at[didx.at[0]],
                                  ssems[b]).wait()

        plsc.subcore_barrier()
        pltpu.sync_copy(acc.at[pl.ds(s * _RPT, _RPT)], stage)
        pltpu.sync_copy(stage, out_hbm.at[c].at[pl.ds(s * _RPT, _RPT)])

    return agg_kernel(table, src2d, dst2d)


def _tc_xw(x_r, W1b):
    def body(x_ref, w_ref, o_ref):
        o_ref[...] = jnp.dot(x_ref[...], w_ref[...],
                             preferred_element_type=jnp.float32)

    return pl.pallas_call(
        body,
        out_shape=jax.ShapeDtypeStruct((_N // 8, 128), jnp.float32),
    )(x_r, W1b)


def _tc_scale(degp, xw_p):
    def body(degp_ref, xw_ref, dis_ref, y1_ref):
        deg = degp_ref[0] + degp_ref[1] + 1.0
        dis = jax.lax.rsqrt(deg)
        dis_ref[...] = dis
        y1_ref[...] = dis[: _N // 8] * xw_ref[...]

    return pl.pallas_call(
        body,
        out_shape=(
            jax.ShapeDtypeStruct((_NPAD // 8, 128), jnp.float32),
            jax.ShapeDtypeStruct((_N // 8, 128), jnp.float32),
        ),
    )(degp.reshape(_NC, _NPAD // 8, 128), xw_p)


def _tc_layer1(agg1p, dis_p, xw_p, b1):
    def body(aggp_ref, dis_ref, xw_ref, b1_ref, h_ref, y2_ref):
        dis = dis_ref[: _N // 8]
        agg = aggp_ref[0, : _N // 8] + aggp_ref[1, : _N // 8]
        pre = dis * agg + dis * dis * xw_ref[...] + b1_ref[...]
        h = jnp.maximum(pre, 0.0)
        h_ref[...] = h
        y2_ref[...] = dis * h

    return pl.pallas_call(
        body,
        out_shape=(
            jax.ShapeDtypeStruct((_N // 8, 128), jnp.float32),
            jax.ShapeDtypeStruct((_N // 8, 128), jnp.float32),
        ),
    )(agg1p.reshape(_NC, _NPAD // 8, 128), dis_p, xw_p,
      jnp.tile(b1, 8).reshape(1, 128))


def _tc_out(agg2p, dis_p, h_p, W2b, b2):
    def body(aggp_ref, dis_ref, h_ref, w2_ref, b2_ref, o_ref):
        dis = dis_ref[: _N // 8]
        agg = aggp_ref[0, : _N // 8] + aggp_ref[1, : _N // 8]
        t = dis * agg + dis * dis * h_ref[...]
        o = jnp.dot(t, w2_ref[...], preferred_element_type=jnp.float32)
        o = o + b2_ref[...]
        parity = jax.lax.broadcasted_iota(jnp.int32, (_N // 8, 16), 1) % 2
        partner = jnp.where(parity == 0,
                            jnp.roll(o, -1, axis=1), jnp.roll(o, 1, axis=1))
        m = jnp.maximum(o, partner)
        lse = m + jnp.log(jnp.exp(o - m) + jnp.exp(partner - m))
        o_ref[...] = o - lse

    return pl.pallas_call(
        body,
        out_shape=jax.ShapeDtypeStruct((_N // 8, 16), jnp.float32),
    )(agg2p.reshape(_NC, _NPAD // 8, 128), dis_p, h_p, W2b,
      jnp.tile(b2, 8).reshape(1, 16))


def kernel(x, edge_index, W1, b1, W2, b2):
    src2d = edge_index[0].astype(jnp.int32).reshape(_NW * _NCHUNK, _K)
    dst2d = edge_index[1].astype(jnp.int32).reshape(_NW * _NCHUNK, _K)
    src2d, dst2d = jax.lax.optimization_barrier((src2d, dst2d))

    eye8 = jnp.eye(8, dtype=jnp.float32)
    W1b = jnp.kron(eye8, W1)
    W2b = jnp.kron(eye8, W2)
    x_r = x.reshape(_N // 8, 8 * _D)

    degp = _sc_degree(dst2d)
    xw_p = _tc_xw(x_r, W1b)
    dis_p, y1_p = _tc_scale(degp, xw_p)
    agg1p = _sc_aggregate(y1_p.reshape(_N, _H), src2d, dst2d)
    h_p, y2_p = _tc_layer1(agg1p, dis_p, xw_p, b1)
    agg2p = _sc_aggregate(y2_p.reshape(_N, _H), src2d, dst2d)
    return _tc_out(agg2p, dis_p, h_p, W2b, b2).reshape(_N, _C)


# speedup vs baseline: 2.0679x; 1.2063x over previous
import functools

import jax
import jax.numpy as jnp
from jax.experimental import pallas as pl
from jax.experimental.pallas import tpu as pltpu
from jax.experimental.pallas import tpu_sc as plsc

_N = 10000
_E = 320000
_D = 128
_H = 16
_C = 2

_NC = 2
_NS = 16
_NW = _NC * _NS
_L = 16

_K = 128
_NROWS = _E // _K
_CBASE = _NROWS // _NW
_CEXTRA = _NROWS - _CBASE * _NW
_NPAD = 10240
_RPT = _NPAD // _NS


def _sc_mesh():
    return plsc.VectorSubcoreMesh(core_axis_name="c", subcore_axis_name="s")


_SC_PARAMS = pltpu.CompilerParams(use_tc_tiling_on_sc=False)


_NBUF = 4
_LAG = 4
_NRING = 2 * _LAG


def _sc_degree(edge3d):
    @functools.partial(
        pl.kernel,
        out_type=jax.ShapeDtypeStruct((_NC, _NPAD, _L), jnp.float32),
        mesh=_sc_mesh(),
        compiler_params=_SC_PARAMS,
        scratch_types=[
            pltpu.VMEM((_CBASE + 1, _K), jnp.int32),
            pltpu.VMEM((_K, _L), jnp.float32),
            pltpu.VMEM((_RPT, _L), jnp.float32),
            pltpu.VMEM_SHARED((_NPAD, _L), jnp.float32),
            pltpu.SemaphoreType.DMA,
        ],
    )
    def deg_kernel(edge_hbm, out_hbm, didx, ones, stage, acc, dsem):
        c = jax.lax.axis_index("c")
        s = jax.lax.axis_index("s")
        wid = s * _NC + c
        extra = wid < _CEXTRA
        nchunk = _CBASE + extra.astype(jnp.int32)
        base = _CBASE * wid + jnp.minimum(wid, _CEXTRA)

        @pl.loop(0, _RPT)
        def _(i):
            stage[i] = jnp.zeros((_L,), jnp.float32)

        @pl.loop(0, _K)
        def _(i):
            ones[i] = jnp.ones((_L,), jnp.float32)

        pltpu.sync_copy(stage, acc.at[pl.ds(s * _RPT, _RPT)])
        pltpu.sync_copy(edge_hbm.at[1].at[pl.ds(base, _CBASE)],
                        didx.at[pl.ds(0, _CBASE)])

        @pl.when(extra)
        def _():
            pltpu.sync_copy(edge_hbm.at[1].at[pl.ds(base + _CBASE, 1)],
                            didx.at[pl.ds(_CBASE, 1)])

        plsc.subcore_barrier()

        def dstep(t):
            pltpu.async_copy(ones, acc.at[didx.at[t]], dsem, add=True)

            @pl.when(t >= 4)
            def _():
                pltpu.make_async_copy(ones, acc.at[didx.at[0]], dsem).wait()

        @pl.loop(0, _CBASE - (_CBASE % 8), step=8)
        def _(g):
            for b in range(8):
                dstep(g + b)

        for t in range(_CBASE - (_CBASE % 8), _CBASE):
            dstep(t)

        @pl.when(extra)
        def _():
            dstep(_CBASE)

        for _i in range(4):
            pltpu.make_async_copy(ones, acc.at[didx.at[0]], dsem).wait()

        plsc.subcore_barrier()
        pltpu.sync_copy(acc.at[pl.ds(s * _RPT, _RPT)], stage)
        pltpu.sync_copy(stage, out_hbm.at[c].at[pl.ds(s * _RPT, _RPT)])

    return deg_kernel(edge3d)


def _sc_aggregate(table, edge3d):
    @functools.partial(
        pl.kernel,
        out_type=jax.ShapeDtypeStruct((_NC, _NPAD, _L), jnp.float32),
        mesh=_sc_mesh(),
        compiler_params=_SC_PARAMS,
        scratch_types=[
            pltpu.VMEM((_CBASE + 1, _K), jnp.int32),
            pltpu.VMEM((_CBASE + 1, _K), jnp.int32),
            pltpu.VMEM((_NRING, _K, _L), jnp.float32),
            pltpu.VMEM((_RPT, _L), jnp.float32),
            pltpu.VMEM_SHARED((_NPAD, _L), jnp.float32),
        ] + [pltpu.SemaphoreType.DMA] * (2 * _NRING),
    )
    def agg_kernel(table_hbm, edge_hbm, out_hbm,
                   sidx, didx, rows, stage, acc, *sems):
        gsems, ssems = sems[:_NRING], sems[_NRING:]
        c = jax.lax.axis_index("c")
        s = jax.lax.axis_index("s")
        wid = s * _NC + c
        extra = wid < _CEXTRA
        nchunk = _CBASE + extra.astype(jnp.int32)
        base = _CBASE * wid + jnp.minimum(wid, _CEXTRA)

        @pl.loop(0, _RPT)
        def _(i):
            stage[i] = jnp.zeros((_L,), jnp.float32)

        pltpu.sync_copy(edge_hbm.at[0].at[pl.ds(base, _CBASE)],
                        sidx.at[pl.ds(0, _CBASE)])
        pltpu.sync_copy(edge_hbm.at[1].at[pl.ds(base, _CBASE)],
                        didx.at[pl.ds(0, _CBASE)])

        @pl.when(extra)
        def _():
            pltpu.sync_copy(edge_hbm.at[0].at[pl.ds(base + _CBASE, 1)],
                            sidx.at[pl.ds(_CBASE, 1)])
            pltpu.sync_copy(edge_hbm.at[1].at[pl.ds(base + _CBASE, 1)],
                            didx.at[pl.ds(_CBASE, 1)])

        pltpu.sync_copy(stage, acc.at[pl.ds(s * _RPT, _RPT)])
        plsc.subcore_barrier()

        def step(t, b):
            bn = (b + _LAG) % _NRING
            pltpu.make_async_copy(
                table_hbm.at[sidx.at[t]], rows.at[b], gsems[b]).wait()
            pltpu.async_copy(rows.at[b], acc.at[didx.at[t]], ssems[b],
                             add=True)

            @pl.when(t >= _LAG)
            def _():
                pltpu.make_async_copy(
                    rows.at[bn], acc.at[didx.at[0]], ssems[bn]).wait()

            @pl.when(t + _LAG < nchunk)
            def _():
                pltpu.async_copy(
                    table_hbm.at[sidx.at[t + _LAG]], rows.at[bn], gsems[bn])

        for b in range(_LAG):
            pltpu.async_copy(table_hbm.at[sidx.at[b]], rows.at[b], gsems[b])

        nmain = _CBASE - (_CBASE % _NRING)

        @pl.loop(0, nmain, step=_NRING)
        def _(g):
            for b in range(_NRING):
                step(g + b, b)

        for t in range(nmain, _CBASE):
            step(t, t % _NRING)

        @pl.when(extra)
        def _():
            step(_CBASE, _CBASE % _NRING)

        for b in range(_NRING):
            @pl.when(jnp.mod(b - (nchunk - _LAG), _NRING) < _LAG)
            def _():
                pltpu.make_async_copy(rows.at[b], acc.at[didx.at[0]],
                                      ssems[b]).wait()

        plsc.subcore_barrier()
        pltpu.sync_copy(acc.at[pl.ds(s * _RPT, _RPT)], stage)
        pltpu.sync_copy(stage, out_hbm.at[c].at[pl.ds(s * _RPT, _RPT)])

    return agg_kernel(table, edge3d)


def _tc_xw(x_r, W1b):
    def body(x_ref, w_ref, o_ref):
        o_ref[...] = jnp.dot(x_ref[...], w_ref[...],
                             preferred_element_type=jnp.float32)

    return pl.pallas_call(
        body,
        out_shape=jax.ShapeDtypeStruct((_N // 8, 128), jnp.float32),
    )(x_r, W1b)


def _tc_scale(degp, xw_p):
    def body(degp_ref, xw_ref, dis_ref, y1_ref):
        deg = degp_ref[0] + degp_ref[1] + 1.0
        dis = jax.lax.rsqrt(deg)
        dis_ref[...] = dis
        y1_ref[...] = dis[: _N // 8] * xw_ref[...]

    return pl.pallas_call(
        body,
        out_shape=(
            jax.ShapeDtypeStruct((_NPAD // 8, 128), jnp.float32),
            jax.ShapeDtypeStruct((_N // 8, 128), jnp.float32),
        ),
    )(degp.reshape(_NC, _NPAD // 8, 128), xw_p)


def _tc_layer1(agg1p, dis_p, xw_p, b1):
    def body(aggp_ref, dis_ref, xw_ref, b1_ref, h_ref, y2_ref):
        dis = dis_ref[: _N // 8]
        agg = aggp_ref[0, : _N // 8] + aggp_ref[1, : _N // 8]
        pre = dis * agg + dis * dis * xw_ref[...] + b1_ref[...]
        h = jnp.maximum(pre, 0.0)
        h_ref[...] = h
        y2_ref[...] = dis * h

    return pl.pallas_call(
        body,
        out_shape=(
            jax.ShapeDtypeStruct((_N // 8, 128), jnp.float32),
            jax.ShapeDtypeStruct((_N // 8, 128), jnp.float32),
        ),
    )(agg1p.reshape(_NC, _NPAD // 8, 128), dis_p, xw_p,
      jnp.tile(b1, 8).reshape(1, 128))


def _tc_out(agg2p, dis_p, h_p, W2b, b2):
    def body(aggp_ref, dis_ref, h_ref, w2_ref, b2_ref, o_ref):
        dis = dis_ref[: _N // 8]
        agg = aggp_ref[0, : _N // 8] + aggp_ref[1, : _N // 8]
        t = dis * agg + dis * dis * h_ref[...]
        o = jnp.dot(t, w2_ref[...], preferred_element_type=jnp.float32)
        o = o + b2_ref[...]
        parity = jax.lax.broadcasted_iota(jnp.int32, (_N // 8, 16), 1) % 2
        partner = jnp.where(parity == 0,
                            jnp.roll(o, -1, axis=1), jnp.roll(o, 1, axis=1))
        m = jnp.maximum(o, partner)
        lse = m + jnp.log(jnp.exp(o - m) + jnp.exp(partner - m))
        o_ref[...] = o - lse

    return pl.pallas_call(
        body,
        out_shape=jax.ShapeDtypeStruct((_N // 8, 16), jnp.float32),
    )(agg2p.reshape(_NC, _NPAD // 8, 128), dis_p, h_p, W2b,
      jnp.tile(b2, 8).reshape(1, 16))


def kernel(x, edge_index, W1, b1, W2, b2):
    edge3d = edge_index.astype(jnp.int32).reshape(2, _NROWS, _K)
    edge3d = jax.lax.optimization_barrier(edge3d)

    eye8 = jnp.eye(8, dtype=jnp.float32)
    W1b = jnp.kron(eye8, W1)
    W2b = jnp.kron(eye8, W2)
    x_r = x.reshape(_N // 8, 8 * _D)

    degp = _sc_degree(edge3d)
    xw_p = _tc_xw(x_r, W1b)
    dis_p, y1_p = _tc_scale(degp, xw_p)
    agg1p = _sc_aggregate(y1_p.reshape(_N, _H), edge3d)
    h_p, y2_p = _tc_layer1(agg1p, dis_p, xw_p, b1)
    agg2p = _sc_aggregate(y2_p.reshape(_N, _H), edge3d)
    return _tc_out(agg2p, dis_p, h_p, W2b, b2).reshape(_N, _C)
